# baseline (speedup 1.0000x reference)
import functools

import jax
import jax.numpy as jnp
from jax import lax
from jax.experimental import pallas as pl
from jax.experimental.pallas import tpu as pltpu
from jax.experimental.pallas import tpu_sc as plsc

B = 4096
DIM = 64
N_OBJ = 17
N_CTX = 16
N_ENT_ROWS = B * 33
N_REL_ROWS = B * 16

NW = 32
EPW = N_ENT_ROWS // NW
RPW = N_REL_ROWS // NW
ECH = 528
RCH = 512

BB = 256
GRID = B // BB


def _gather_chunk(table, idx_hbm, out_hbm, idx_v, rows_v, sem, src_base,
                  dst_base, n, lanes):
    pltpu.sync_copy(idx_hbm.at[pl.ds(src_base, n)], idx_v.at[pl.ds(0, n)])

    def vec(v16, _):
        iv = idx_v[pl.ds(v16 * 16, 16)]

        def row(l, _):
            r = jnp.sum(jnp.where(lanes == l, iv, 0))
            pltpu.async_copy(table.at[pl.ds(r, 1)],
                             rows_v.at[pl.ds(v16 * 16 + l, 1)], sem)
            return 0

        lax.fori_loop(0, 16, row, 0)
        return 0

    lax.fori_loop(0, n // 16, vec, 0)

    def drain(i, _):
        pltpu.make_async_copy(table.at[pl.ds(0, 1)],
                              rows_v.at[pl.ds(i, 1)], sem).wait()
        return 0

    lax.fori_loop(0, n, drain, 0)
    pltpu.sync_copy(rows_v.at[pl.ds(0, n)], out_hbm.at[pl.ds(dst_base, n)])


def _sc_body(ent_emb, rel_emb, ent_idx, rel_idx, out_ent, out_rel,
             idx_v, rows_v, sem):
    wid = lax.axis_index("s") * 2 + lax.axis_index("c")
    lanes = lax.broadcasted_iota(jnp.int32, (16,), 0)
    ebase = wid * EPW
    rbase = wid * RPW

    def ent_chunk(c, _):
        _gather_chunk(ent_emb, ent_idx, out_ent, idx_v, rows_v, sem,
                      ebase + c * ECH, ebase + c * ECH, ECH, lanes)
        return 0

    lax.fori_loop(0, EPW // ECH, ent_chunk, 0)

    def rel_chunk(c, _):
        _gather_chunk(rel_emb, rel_idx, out_rel, idx_v, rows_v, sem,
                      rbase + c * RCH, rbase + c * RCH, RCH, lanes)
        return 0

    lax.fori_loop(0, RPW // RCH, rel_chunk, 0)


_sc_gather = functools.partial(
    pl.kernel,
    out_type=(
        jax.ShapeDtypeStruct((N_ENT_ROWS, DIM), jnp.float32),
        jax.ShapeDtypeStruct((N_REL_ROWS, DIM), jnp.float32),
    ),
    mesh=plsc.VectorSubcoreMesh(core_axis_name="c", subcore_axis_name="s"),
    scratch_types=(
        pltpu.VMEM((ECH,), jnp.int32),
        pltpu.VMEM((ECH, DIM), jnp.float32),
        pltpu.SemaphoreType.DMA,
    ),
    compiler_params=pltpu.CompilerParams(needs_layout_passes=False),
)(_sc_body)


def _tc_body(ent_ref, rel_ref, out_ref):
    e = ent_ref[...]
    r = rel_ref[...]
    obj = e[:, :N_OBJ, :]
    obj2 = jnp.concatenate([obj, obj], axis=2)
    hh = e[:, 17:25, :] + r[:, :8, :]
    tt = e[:, 25:33, :] - r[:, 8:16, :]
    ctx2 = jnp.concatenate([hh, tt], axis=2)
    acc = jnp.zeros((BB, N_OBJ), jnp.float32)
    for k in range(8):
        d = obj2 - ctx2[:, k:k + 1, :]
        s = d * d
        acc = acc + jnp.sqrt(jnp.sum(s[:, :, :DIM], axis=2)) \
                  + jnp.sqrt(jnp.sum(s[:, :, DIM:], axis=2))
    f1 = acc * (-1.0 / N_CTX)
    col = lax.broadcasted_iota(jnp.int32, (BB, N_OBJ), 1)
    x = jnp.where(col == 0, f1, -f1)
    ls = jnp.minimum(x, 0.0) - jnp.log1p(jnp.exp(-jnp.abs(x)))
    bs = jnp.sum(ls)

    @pl.when(pl.program_id(0) == 0)
    def _():
        out_ref[0, 0] = 0.0

    out_ref[0, 0] += bs


_tc_reduce = pl.pallas_call(
    _tc_body,
    grid=(GRID,),
    in_specs=[
        pl.BlockSpec((BB, 33, DIM), lambda i: (i, 0, 0)),
        pl.BlockSpec((BB, 16, DIM), lambda i: (i, 0, 0)),
    ],
    out_specs=pl.BlockSpec((1, 1), lambda i: (0, 0),
                           memory_space=pltpu.SMEM),
    out_shape=jax.ShapeDtypeStruct((1, 1), jnp.float32),
    compiler_params=pltpu.CompilerParams(
        dimension_semantics=("arbitrary",)),
)


def kernel(entity_batch, head_batch, head_relation_batch,
           tail_relation_batch, tail_batch, negative_batch,
           entity_emb, relation_emb):
    obj_idx = jnp.concatenate([entity_batch[:, None], negative_batch], axis=1)
    ent_idx = jnp.concatenate([obj_idx, head_batch, tail_batch], axis=1)
    ent_idx = ent_idx.astype(jnp.int32).reshape(-1)
    rel_idx = jnp.concatenate([head_relation_batch, tail_relation_batch],
                              axis=1)
    rel_idx = rel_idx.astype(jnp.int32).reshape(-1)

    out_ent, out_rel = _sc_gather(entity_emb, relation_emb, ent_idx, rel_idx)
    total = _tc_reduce(out_ent.reshape(B, 33, DIM),
                       out_rel.reshape(B, 16, DIM))
    return total[0, 0]


# speedup vs baseline: 1.4919x; 1.4919x over previous
import functools

import jax
import jax.numpy as jnp
from jax import lax
from jax.experimental import pallas as pl
from jax.experimental.pallas import tpu as pltpu
from jax.experimental.pallas import tpu_sc as plsc

B = 4096
DIM = 64
N_OBJ = 17
N_CTX = 16
N_ENT_ROWS = B * 33
N_REL_ROWS = B * 16

NW = 32
EPW = N_ENT_ROWS // NW
RPW = N_REL_ROWS // NW
ECH = 528
RCH = 512

BB = 128
NP = 24
GRID = B // BB


def _gather_chunk(table, idx_hbm, out_hbm, idx_v, rows_v, sem, src_base,
                  dst_base, n, lanes):
    pltpu.sync_copy(idx_hbm.at[pl.ds(src_base, n)], idx_v.at[pl.ds(0, n)])

    def vec(v16, _):
        iv = idx_v[pl.ds(v16 * 16, 16)]

        def row(l, _):
            r = jnp.sum(jnp.where(lanes == l, iv, 0))
            pltpu.async_copy(table.at[pl.ds(r, 1)],
                             rows_v.at[pl.ds(v16 * 16 + l, 1)], sem)
            return 0

        lax.fori_loop(0, 16, row, 0)
        return 0

    lax.fori_loop(0, n // 16, vec, 0)

    def drain(i, _):
        pltpu.make_async_copy(table.at[pl.ds(0, 1)],
                              rows_v.at[pl.ds(i, 1)], sem).wait()
        return 0

    lax.fori_loop(0, n, drain, 0)
    pltpu.sync_copy(rows_v.at[pl.ds(0, n)], out_hbm.at[pl.ds(dst_base, n)])


def _sc_body(ent_emb, rel_emb, ent_idx, rel_idx, out_ent, out_rel,
             idx_v, rows_v, sem):
    wid = lax.axis_index("s") * 2 + lax.axis_index("c")
    lanes = lax.broadcasted_iota(jnp.int32, (16,), 0)
    ebase = wid * EPW
    rbase = wid * RPW

    def ent_chunk(c, _):
        _gather_chunk(ent_emb, ent_idx, out_ent, idx_v, rows_v, sem,
                      ebase + c * ECH, ebase + c * ECH, ECH, lanes)
        return 0

    lax.fori_loop(0, EPW // ECH, ent_chunk, 0)

    def rel_chunk(c, _):
        _gather_chunk(rel_emb, rel_idx, out_rel, idx_v, rows_v, sem,
                      rbase + c * RCH, rbase + c * RCH, RCH, lanes)
        return 0

    lax.fori_loop(0, RPW // RCH, rel_chunk, 0)


_sc_gather = functools.partial(
    pl.kernel,
    out_type=(
        jax.ShapeDtypeStruct((N_ENT_ROWS, DIM), jnp.float32),
        jax.ShapeDtypeStruct((N_REL_ROWS, DIM), jnp.float32),
    ),
    mesh=plsc.VectorSubcoreMesh(core_axis_name="c", subcore_axis_name="s"),
    scratch_types=(
        pltpu.VMEM((ECH,), jnp.int32),
        pltpu.VMEM((ECH, DIM), jnp.float32),
        pltpu.SemaphoreType.DMA,
    ),
    compiler_params=pltpu.CompilerParams(needs_layout_passes=False),
)(_sc_body)


def _tc_body(ent_ref, rel_ref, w_ref, out_ref, scat_ref):
    e = ent_ref[...]
    r = rel_ref[...]
    obj = e[:, :N_OBJ, :]
    obj2 = jnp.concatenate([obj, obj], axis=2)
    obj2p = jnp.concatenate(
        [obj2, jnp.zeros((BB, NP - N_OBJ, 2 * DIM), jnp.float32)], axis=1)
    hh = e[:, 17:25, :] + r[:, :8, :]
    tt = e[:, 25:33, :] - r[:, 8:16, :]
    ctx2 = jnp.concatenate([hh, tt], axis=2)
    for k in range(8):
        d = obj2p - ctx2[:, k:k + 1, :]
        scat_ref[:, k * 128:(k + 1) * 128] = (d * d).reshape(BB * NP, 128)
    d2 = jax.lax.dot_general(scat_ref[...], w_ref[...],
                             (((1,), (0,)), ((), ())),
                             preferred_element_type=jnp.float32)
    acc = jnp.sum(jnp.sqrt(d2), axis=1)
    f1 = acc * (-1.0 / N_CTX)
    pos = lax.broadcasted_iota(jnp.int32, (BB * NP,), 0)
    x = jnp.where(pos % NP == 0, f1, -f1)
    ls = jnp.minimum(x, 0.0) - jnp.log1p(jnp.exp(-jnp.abs(x)))
    bs = jnp.sum(jnp.where(pos % NP < N_OBJ, ls, 0.0))

    @pl.when(pl.program_id(0) == 0)
    def _():
        out_ref[0, 0] = 0.0

    out_ref[0, 0] += bs


_tc_reduce = pl.pallas_call(
    _tc_body,
    grid=(GRID,),
    in_specs=[
        pl.BlockSpec((BB, 33, DIM), lambda i: (i, 0, 0)),
        pl.BlockSpec((BB, 16, DIM), lambda i: (i, 0, 0)),
        pl.BlockSpec((8 * 128, N_CTX), lambda i: (0, 0)),
    ],
    out_specs=pl.BlockSpec((1, 1), lambda i: (0, 0),
                           memory_space=pltpu.SMEM),
    out_shape=jax.ShapeDtypeStruct((1, 1), jnp.float32),
    scratch_shapes=[pltpu.VMEM((BB * NP, 8 * 128), jnp.float32)],
    compiler_params=pltpu.CompilerParams(
        dimension_semantics=("arbitrary",)),
)


def kernel(entity_batch, head_batch, head_relation_batch,
           tail_relation_batch, tail_batch, negative_batch,
           entity_emb, relation_emb):
    obj_idx = jnp.concatenate([entity_batch[:, None], negative_batch], axis=1)
    ent_idx = jnp.concatenate([obj_idx, head_batch, tail_batch], axis=1)
    ent_idx = ent_idx.astype(jnp.int32).reshape(-1)
    rel_idx = jnp.concatenate([head_relation_batch, tail_relation_batch],
                              axis=1)
    rel_idx = rel_idx.astype(jnp.int32).reshape(-1)

    rows = jnp.arange(8 * 128, dtype=jnp.int32)
    tgt = 2 * (rows // 128) + ((rows % 128) >= DIM).astype(jnp.int32)
    w = (tgt[:, None] == jnp.arange(N_CTX, dtype=jnp.int32)[None, :]) \
        .astype(jnp.float32)

    out_ent, out_rel = _sc_gather(entity_emb, relation_emb, ent_idx, rel_idx)
    total = _tc_reduce(out_ent.reshape(B, 33, DIM),
                       out_rel.reshape(B, 16, DIM), w)
    return total[0, 0]


# speedup vs baseline: 1.5868x; 1.0636x over previous
import functools

import jax
import jax.numpy as jnp
from jax import lax
from jax.experimental import pallas as pl
from jax.experimental.pallas import tpu as pltpu
from jax.experimental.pallas import tpu_sc as plsc

B = 4096
DIM = 64
N_OBJ = 17
N_CTX = 16
S_ENT = 33
N_ENT_ROWS = B * S_ENT
N_REL_ROWS = B * N_CTX

NW = 32
EPW = N_ENT_ROWS // NW
RPW = N_REL_ROWS // NW
ECH = 528
RCH = 512

BB = 256
GRID = B // BB


def _gather_chunk(table, idx_hbm, out_hbm, idx_v, rows_v, sem, base, n,
                  lanes):
    pltpu.sync_copy(idx_hbm.at[pl.ds(base, n)], idx_v.at[pl.ds(0, n)])

    def vec(v16, _):
        iv = idx_v[pl.ds(v16 * 16, 16)]

        def row(l, _):
            r = jnp.sum(jnp.where(lanes == l, iv, 0))
            pltpu.async_copy(table.at[pl.ds(r, 1)],
                             rows_v.at[pl.ds(v16 * 16 + l, 1)], sem)
            return 0

        lax.fori_loop(0, 16, row, 0)
        return 0

    lax.fori_loop(0, n // 16, vec, 0)

    def drain(i, _):
        pltpu.make_async_copy(table.at[pl.ds(0, 1)],
                              rows_v.at[pl.ds(i, 1)], sem).wait()
        return 0

    lax.fori_loop(0, n, drain, 0)
    pltpu.sync_copy(rows_v.at[pl.ds(0, n)], out_hbm.at[pl.ds(base, n)])


def _sc_body(ent_emb, rel_emb, ent_idx, rel_idx, out_ent, out_rel,
             idx_v, rows_v, sem):
    wid = lax.axis_index("s") * 2 + lax.axis_index("c")
    lanes = lax.broadcasted_iota(jnp.int32, (16,), 0)
    ebase = wid * EPW
    rbase = wid * RPW

    def ent_chunk(c, _):
        _gather_chunk(ent_emb, ent_idx, out_ent, idx_v, rows_v, sem,
                      ebase + c * ECH, ECH, lanes)
        return 0

    lax.fori_loop(0, EPW // ECH, ent_chunk, 0)

    def rel_chunk(c, _):
        _gather_chunk(rel_emb, rel_idx, out_rel, idx_v, rows_v, sem,
                      rbase + c * RCH, RCH, lanes)
        return 0

    lax.fori_loop(0, RPW // RCH, rel_chunk, 0)


_sc_gather = functools.partial(
    pl.kernel,
    out_type=(
        jax.ShapeDtypeStruct((N_ENT_ROWS, DIM), jnp.float32),
        jax.ShapeDtypeStruct((N_REL_ROWS, DIM), jnp.float32),
    ),
    mesh=plsc.VectorSubcoreMesh(core_axis_name="c", subcore_axis_name="s"),
    scratch_types=(
        pltpu.VMEM((ECH,), jnp.int32),
        pltpu.VMEM((ECH, DIM), jnp.float32),
        pltpu.SemaphoreType.DMA,
    ),
    compiler_params=pltpu.CompilerParams(needs_layout_passes=False),
)(_sc_body)


def _tc_body(ent_ref, rel_ref, w_ref, out_ref, scat_ref):
    e3 = ent_ref[...]
    r3 = rel_ref[...]
    obj = e3[:N_OBJ]
    hh = e3[17:25] + r3[:8]
    tt = e3[25:33] - r3[8:16]
    for k in range(8):
        d = obj - hh[k]
        scat_ref[:, k * DIM:(k + 1) * DIM] = (d * d).reshape(
            N_OBJ * BB, DIM)
        d = obj - tt[k]
        scat_ref[:, (8 + k) * DIM:(9 + k) * DIM] = (d * d).reshape(
            N_OBJ * BB, DIM)
    d2 = jax.lax.dot_general(scat_ref[...], w_ref[...],
                             (((1,), (0,)), ((), ())),
                             preferred_element_type=jnp.float32)
    acc = jnp.sum(jnp.sqrt(d2), axis=1)
    f1 = acc * (-1.0 / N_CTX)
    pos = lax.broadcasted_iota(jnp.int32, (N_OBJ * BB,), 0)
    x = jnp.where(pos < BB, f1, -f1)
    ls = jnp.minimum(x, 0.0) - jnp.log1p(jnp.exp(-jnp.abs(x)))
    bs = jnp.sum(ls)

    @pl.when(pl.program_id(0) == 0)
    def _():
        out_ref[0, 0] = 0.0

    out_ref[0, 0] += bs


_tc_reduce = pl.pallas_call(
    _tc_body,
    grid=(GRID,),
    in_specs=[
        pl.BlockSpec((S_ENT, BB, DIM), lambda i: (0, i, 0)),
        pl.BlockSpec((N_CTX, BB, DIM), lambda i: (0, i, 0)),
        pl.BlockSpec((N_CTX * DIM, N_CTX), lambda i: (0, 0)),
    ],
    out_specs=pl.BlockSpec((1, 1), lambda i: (0, 0),
                           memory_space=pltpu.SMEM),
    out_shape=jax.ShapeDtypeStruct((1, 1), jnp.float32),
    scratch_shapes=[pltpu.VMEM((N_OBJ * BB, N_CTX * DIM), jnp.float32)],
    compiler_params=pltpu.CompilerParams(
        dimension_semantics=("arbitrary",)),
)


def kernel(entity_batch, head_batch, head_relation_batch,
           tail_relation_batch, tail_batch, negative_batch,
           entity_emb, relation_emb):
    obj_idx = jnp.concatenate([entity_batch[:, None], negative_batch], axis=1)
    ent_idx = jnp.concatenate([obj_idx, head_batch, tail_batch], axis=1)
    ent_idx = ent_idx.astype(jnp.int32).T.reshape(-1)
    rel_idx = jnp.concatenate([head_relation_batch, tail_relation_batch],
                              axis=1)
    rel_idx = rel_idx.astype(jnp.int32).T.reshape(-1)

    rows = jnp.arange(N_CTX * DIM, dtype=jnp.int32) // DIM
    w = (rows[:, None] == jnp.arange(N_CTX, dtype=jnp.int32)[None, :]) \
        .astype(jnp.float32)

    out_ent, out_rel = _sc_gather(entity_emb, relation_emb, ent_idx, rel_idx)
    total = _tc_reduce(out_ent.reshape(S_ENT, B, DIM),
                       out_rel.reshape(N_CTX, B, DIM), w)
    return total[0, 0]
